# baseline (speedup 1.0000x reference)
import functools

import jax
import jax.numpy as jnp
from jax import lax
from jax.experimental import pallas as pl
from jax.experimental.pallas import tpu as pltpu
from jax.experimental.pallas import tpu_sc as plsc

EMBED_DIM = 64
NC = 2
NS = 16
NW = NC * NS
CHUNK = 128


def _make_gather(batch: int, n_rows: int, n_chunks: int):
    b_per_w = n_chunks * CHUNK
    mesh = plsc.VectorSubcoreMesh(core_axis_name="c", subcore_axis_name="s")

    @functools.partial(
        pl.kernel,
        mesh=mesh,
        out_type=jax.ShapeDtypeStruct((batch, EMBED_DIM), jnp.float32),
        scratch_types=[
            pltpu.VMEM((n_chunks, CHUNK), jnp.int32),
            pltpu.VMEM((CHUNK, EMBED_DIM), jnp.float32),
            pltpu.SemaphoreType.DMA,
        ],
        compiler_params=pltpu.CompilerParams(use_tc_tiling_on_sc=False),
    )
    def gather_kernel(tokens_hbm, table_hbm, out_hbm, idx_v, rows_v, sem):
        wid = lax.axis_index("s") * NC + lax.axis_index("c")
        base = wid * b_per_w
        pltpu.sync_copy(tokens_hbm.at[wid], idx_v)

        def body(j, carry):
            pltpu.async_copy(table_hbm.at[idx_v.at[j]], rows_v, sem).wait()
            pltpu.sync_copy(rows_v, out_hbm.at[pl.ds(base + j * CHUNK, CHUNK)])
            return carry

        lax.fori_loop(0, n_chunks, body, 0)

    return gather_kernel


def kernel(tokens, table):
    orig_shape = tokens.shape
    batch = tokens.size
    assert batch % (NW * CHUNK) == 0
    n_chunks = batch // (NW * CHUNK)
    idx = tokens.reshape(NW, n_chunks, CHUNK).astype(jnp.int32)
    out = _make_gather(batch, table.shape[0], n_chunks)(idx, table)
    return out.reshape(*orig_shape, EMBED_DIM)


# speedup vs baseline: 1.0514x; 1.0514x over previous
import functools

import jax
import jax.numpy as jnp
from jax import lax
from jax.experimental import pallas as pl
from jax.experimental.pallas import tpu as pltpu
from jax.experimental.pallas import tpu_sc as plsc

EMBED_DIM = 64
NC = 2
NS = 16
NW = NC * NS
BLK = 128


def _make_gather(n_batch: int, n_seq: int):
    mesh = plsc.VectorSubcoreMesh(core_axis_name="c", subcore_axis_name="s")

    @functools.partial(
        pl.kernel,
        mesh=mesh,
        out_type=jax.ShapeDtypeStruct((n_batch, n_seq * EMBED_DIM), jnp.float32),
        scratch_types=[
            pltpu.VMEM((n_seq, BLK), jnp.int32),
            pltpu.VMEM((BLK, EMBED_DIM), jnp.float32),
            pltpu.SemaphoreType.DMA,
        ],
        compiler_params=pltpu.CompilerParams(use_tc_tiling_on_sc=False),
    )
    def gather_kernel(tokens_t_hbm, table_hbm, out_hbm, idx_v, rows_v, sem):
        wid = lax.axis_index("s") * NC + lax.axis_index("c")
        base = wid * BLK
        pltpu.sync_copy(tokens_t_hbm.at[:, pl.ds(base, BLK)], idx_v)

        def body(j, carry):
            pltpu.async_copy(table_hbm.at[idx_v.at[j]], rows_v, sem).wait()
            pltpu.sync_copy(
                rows_v, out_hbm.at[pl.ds(base, BLK), pl.ds(j * EMBED_DIM, EMBED_DIM)]
            )
            return carry

        lax.fori_loop(0, n_seq, body, 0)

    return gather_kernel


def kernel(tokens, table):
    n_batch, n_seq = tokens.shape
    assert n_batch % NW == 0 and n_batch // NW == BLK
    tokens_t = tokens.T.astype(jnp.int32)
    out = _make_gather(n_batch, n_seq)(tokens_t, table)
    return out.reshape(n_batch, n_seq, EMBED_DIM)
